# baseline (speedup 1.0000x reference)
import functools

import jax
import jax.numpy as jnp
from jax import lax
from jax.experimental import pallas as pl
from jax.experimental.pallas import tpu as pltpu
from jax.experimental.pallas import tpu_sc as plsc

NUM_TOKENS = 8192
DIM = 768

_info = plsc.get_sparse_core_info()
_NC = _info.num_cores
_NS = _info.num_subcores
_NW = _NC * _NS
_ROWS_PER_W = NUM_TOKENS // _NW


@functools.partial(
    pl.kernel,
    out_type=jax.ShapeDtypeStruct((NUM_TOKENS, DIM), jnp.float32),
    mesh=plsc.VectorSubcoreMesh(core_axis_name="c", subcore_axis_name="s"),
)
def _identity_rows_sc(table_hbm, out_hbm):
    wid = lax.axis_index("s") * _NC + lax.axis_index("c")
    base = wid * _ROWS_PER_W
    pltpu.sync_copy(
        table_hbm.at[pl.ds(base, _ROWS_PER_W)],
        out_hbm.at[pl.ds(base, _ROWS_PER_W)],
    )


def kernel(table):
    out = _identity_rows_sc(table)
    return out[:, None, :]


# speedup vs baseline: 13.7538x; 13.7538x over previous
import functools

import jax
import jax.numpy as jnp
from jax import lax
from jax.experimental import pallas as pl
from jax.experimental.pallas import tpu as pltpu
from jax.experimental.pallas import tpu_sc as plsc

NUM_TOKENS = 8192
DIM = 768

_info = plsc.get_sparse_core_info()
_NC = _info.num_cores
_NS = _info.num_subcores
_NW = _NC * _NS
_ROWS_PER_W = NUM_TOKENS // _NW
_CH = 32
_NCHUNK = _ROWS_PER_W // _CH
_NBUF = 4


@functools.partial(
    pl.kernel,
    out_type=jax.ShapeDtypeStruct((NUM_TOKENS, DIM), jnp.float32),
    mesh=plsc.VectorSubcoreMesh(core_axis_name="c", subcore_axis_name="s"),
    scratch_types=(
        [pltpu.VMEM((_NBUF, _CH, DIM), jnp.float32)]
        + [pltpu.SemaphoreType.DMA] * (2 * _NBUF)
    ),
)
def _identity_rows_sc(table_hbm, out_hbm, buf, *sems):
    in_sems, out_sems = sems[:_NBUF], sems[_NBUF:]
    wid = lax.axis_index("s") * _NC + lax.axis_index("c")
    base = wid * _ROWS_PER_W

    def start_in(c, b):
        return pltpu.async_copy(
            table_hbm.at[pl.ds(base + c * _CH, _CH)], buf.at[b], in_sems[b])

    def start_out(c, b):
        return pltpu.async_copy(
            buf.at[b], out_hbm.at[pl.ds(base + c * _CH, _CH)], out_sems[b])

    in_h = [start_in(b, b) for b in range(_NBUF)]
    tail = []
    for c in range(_NCHUNK):
        b = c % _NBUF
        in_h[b].wait()
        out_h = start_out(c, b)
        nxt = c + _NBUF
        if nxt < _NCHUNK:
            out_h.wait()
            in_h[b] = start_in(nxt, b)
        else:
            tail.append(out_h)
    for h in tail:
        h.wait()


def kernel(table):
    out = _identity_rows_sc(table)
    return out[:, None, :]


# speedup vs baseline: 13.7782x; 1.0018x over previous
import functools

import jax
import jax.numpy as jnp
from jax import lax
from jax.experimental import pallas as pl
from jax.experimental.pallas import tpu as pltpu
from jax.experimental.pallas import tpu_sc as plsc

NUM_TOKENS = 8192
DIM = 768

_info = plsc.get_sparse_core_info()
_NC = _info.num_cores
_NS = _info.num_subcores
_NW = _NC * _NS
_ROWS_PER_W = NUM_TOKENS // _NW
_RA = 128
_RB = 32
_RC = _ROWS_PER_W - _RA - _RB


@functools.partial(
    pl.kernel,
    out_type=jax.ShapeDtypeStruct((NUM_TOKENS, DIM), jnp.float32),
    mesh=plsc.VectorSubcoreMesh(core_axis_name="c", subcore_axis_name="s"),
    scratch_types=(
        [pltpu.VMEM((_RA, DIM), jnp.float32),
         pltpu.VMEM_SHARED((_NS, _RB, DIM), jnp.float32)]
        + [pltpu.SemaphoreType.DMA] * 6
    ),
)
def _identity_rows_sc(table_hbm, out_hbm, buf_a, buf_b,
                      sa_in, sb_in, sc_in, sa_out, sb_out, sc_out):
    sid = lax.axis_index("s")
    wid = sid * _NC + lax.axis_index("c")
    base = wid * _ROWS_PER_W
    base_b = base + _RA
    base_c = base_b + _RB

    in_a = pltpu.async_copy(table_hbm.at[pl.ds(base, _RA)], buf_a, sa_in)
    in_b = pltpu.async_copy(
        table_hbm.at[pl.ds(base_b, _RB)], buf_b.at[sid], sb_in)
    in_a.wait()
    out_a = pltpu.async_copy(buf_a, out_hbm.at[pl.ds(base, _RA)], sa_out)
    in_b.wait()
    out_b = pltpu.async_copy(
        buf_b.at[sid], out_hbm.at[pl.ds(base_b, _RB)], sb_out)
    out_a.wait()
    in_c = pltpu.async_copy(
        table_hbm.at[pl.ds(base_c, _RC)], buf_a.at[pl.ds(0, _RC)], sc_in)
    in_c.wait()
    out_c = pltpu.async_copy(
        buf_a.at[pl.ds(0, _RC)], out_hbm.at[pl.ds(base_c, _RC)], sc_out)
    out_b.wait()
    out_c.wait()


def kernel(table):
    out = _identity_rows_sc(table)
    return out[:, None, :]


# speedup vs baseline: 21.9956x; 1.5964x over previous
import functools

import jax
import jax.numpy as jnp
from jax import lax
from jax.experimental import pallas as pl
from jax.experimental.pallas import tpu as pltpu
from jax.experimental.pallas import tpu_sc as plsc

NUM_TOKENS = 8192
DIM = 768

_info = plsc.get_sparse_core_info()
_NC = _info.num_cores
_NS = _info.num_subcores
_NW = _NC * _NS
_ROWS_PER_W = NUM_TOKENS // _NW
_RA = 128
_RB = 32
_RC = _ROWS_PER_W - _RA - _RB


@functools.partial(
    pl.kernel,
    out_type=jax.ShapeDtypeStruct((NUM_TOKENS, 1, DIM), jnp.float32),
    mesh=plsc.VectorSubcoreMesh(core_axis_name="c", subcore_axis_name="s"),
    scratch_types=(
        [pltpu.VMEM((_RA, DIM), jnp.float32),
         pltpu.VMEM_SHARED((_NS, _RB, DIM), jnp.float32)]
        + [pltpu.SemaphoreType.DMA] * 6
    ),
)
def _identity_rows_sc(table_hbm, out_hbm, buf_a, buf_b,
                      sa_in, sb_in, sc_in, sa_out, sb_out, sc_out):
    sid = lax.axis_index("s")
    wid = sid * _NC + lax.axis_index("c")
    base = wid * _ROWS_PER_W
    base_b = base + _RA
    base_c = base_b + _RB

    in_a = pltpu.async_copy(table_hbm.at[pl.ds(base, _RA)], buf_a, sa_in)
    in_b = pltpu.async_copy(
        table_hbm.at[pl.ds(base_b, _RB)], buf_b.at[sid], sb_in)
    in_a.wait()
    out_a = pltpu.async_copy(
        buf_a, out_hbm.at[pl.ds(base, _RA), 0], sa_out)
    in_b.wait()
    out_b = pltpu.async_copy(
        buf_b.at[sid], out_hbm.at[pl.ds(base_b, _RB), 0], sb_out)
    out_a.wait()
    in_c = pltpu.async_copy(
        table_hbm.at[pl.ds(base_c, _RC)], buf_a.at[pl.ds(0, _RC)], sc_in)
    in_c.wait()
    out_c = pltpu.async_copy(
        buf_a.at[pl.ds(0, _RC)], out_hbm.at[pl.ds(base_c, _RC), 0], sc_out)
    out_b.wait()
    out_c.wait()


def kernel(table):
    return _identity_rows_sc(table)
